# baseline (speedup 1.0000x reference)
import functools

import jax
import jax.numpy as jnp
from jax import lax
from jax.experimental import pallas as pl
from jax.experimental.pallas import tpu as pltpu
from jax.experimental.pallas import tpu_sc as plsc

D = 128
F = 64
TPAD = 72
N = 8192
ROWS_PER_STEP = 8192

NC, NS = 2, 16
NW = NC * NS
B_PER_W = N // NW
CHUNK = 64


def _tc_body(mask_ref, emb_ref, w1_ref, b1_ref, w2_ref, b2_ref,
             counts_ref, table_ref):
    m = mask_ref[...]
    rows = m.shape[0]
    q = jnp.dot(m.astype(jnp.float32), jnp.ones((F, D), jnp.float32),
                preferred_element_type=jnp.float32)
    q3 = q.reshape(rows // D, D, D)
    lane = lax.broadcasted_iota(jnp.int32, q3.shape, 2)
    sub = lax.broadcasted_iota(jnp.int32, q3.shape, 1)
    eye = (lane == sub).astype(jnp.float32)
    counts_ref[...] = jnp.sum(q3 * eye, axis=1).astype(jnp.int32)

    @pl.when(pl.program_id(0) == 0)
    def _():
        r = lax.broadcasted_iota(jnp.int32, (TPAD, D), 0).astype(
            jnp.float32) * (1.0 / F)
        x = r * w1_ref[...] + b1_ref[...]
        h = x * jax.nn.sigmoid(x)
        ratio = jnp.dot(h, w2_ref[...], preferred_element_type=jnp.float32)
        e0 = emb_ref[0:1, :]
        e1 = emb_ref[1:2, :]
        table_ref[...] = ratio + b2_ref[...] + e0 + r * (e1 - e0)


def _counts_and_table(mask2d, emb_table, W1, b1, W2, b2,
                      rows_per_step=ROWS_PER_STEP):
    return pl.pallas_call(
        _tc_body,
        grid=(N // rows_per_step,),
        in_specs=[
            pl.BlockSpec((rows_per_step, F), lambda i: (i, 0)),
            pl.BlockSpec((2, D), lambda i: (0, 0)),
            pl.BlockSpec((1, D), lambda i: (0, 0)),
            pl.BlockSpec((1, D), lambda i: (0, 0)),
            pl.BlockSpec((D, D), lambda i: (0, 0)),
            pl.BlockSpec((1, D), lambda i: (0, 0)),
        ],
        out_specs=[
            pl.BlockSpec((rows_per_step // D, D), lambda i: (i, 0)),
            pl.BlockSpec((TPAD, D), lambda i: (0, 0)),
        ],
        out_shape=[
            jax.ShapeDtypeStruct((N // D, D), jnp.int32),
            jax.ShapeDtypeStruct((TPAD, D), jnp.float32),
        ],
    )(mask2d, emb_table, W1, b1, W2, b2)


NCHUNK = B_PER_W // CHUNK


def _sc_gather_body(table_hbm, counts_hbm, out_hbm, table_sp, idx_v,
                    rows, semg, semw):
    sid = lax.axis_index("s")
    wid = sid * NC + lax.axis_index("c")

    @pl.when(sid == 0)
    def _():
        pltpu.sync_copy(table_hbm, table_sp)

    pltpu.sync_copy(counts_hbm.at[pl.ds(wid * NCHUNK, NCHUNK)], idx_v)
    plsc.subcore_barrier()
    gathers = [
        pltpu.async_copy(table_sp.at[idx_v.at[k]], rows.at[k], semg)
        for k in range(NCHUNK)
    ]
    writes = []
    for k in range(NCHUNK):
        gathers[k].wait()
        writes.append(pltpu.async_copy(
            rows.at[k], out_hbm.at[pl.ds(wid * B_PER_W + k * CHUNK, CHUNK)],
            semw))
    for w in writes:
        w.wait()


@functools.cache
def _sc_gather():
    return pl.kernel(
        _sc_gather_body,
        out_type=jax.ShapeDtypeStruct((N, D), jnp.float32),
        mesh=plsc.VectorSubcoreMesh(core_axis_name="c", subcore_axis_name="s",
                                    num_cores=NC, num_subcores=NS),
        scratch_types=[
            pltpu.VMEM_SHARED((TPAD, D), jnp.float32),
            pltpu.VMEM((NCHUNK, CHUNK), jnp.int32),
            pltpu.VMEM((NCHUNK, CHUNK, D), jnp.float32),
            pltpu.SemaphoreType.DMA,
            pltpu.SemaphoreType.DMA,
        ],
    )


def kernel(mask, emb_table, W1, b1, W2, b2):
    Bsz, S, _ = mask.shape
    mask2d = mask.reshape(N, F)
    counts, table = _counts_and_table(
        mask2d, emb_table, W1, b1.reshape(1, D), W2, b2.reshape(1, D))
    out = _sc_gather()(table, counts.reshape(N // CHUNK, CHUNK))
    return out.reshape(Bsz, S, D)


# speedup vs baseline: 1.0542x; 1.0542x over previous
import functools

import jax
import jax.numpy as jnp
from jax import lax
from jax.experimental import pallas as pl
from jax.experimental.pallas import tpu as pltpu
from jax.experimental.pallas import tpu_sc as plsc

D = 128
F = 64
TPAD = 72
N = 8192

NC, NS = 2, 16
NW = NC * NS
B_PER_W = N // NW
CHUNK = 128


def _tc_body(mask_ref, emb_ref, w1_ref, b1_ref, w2_ref, b2_ref,
             counts_ref, table_ref):
    m = mask_ref[...]
    rows = m.shape[0]
    f = lax.broadcasted_iota(jnp.int32, (D, D), 0)
    l2 = lax.broadcasted_iota(jnp.int32, (D, D), 1)
    blockdiag = ((f < F) == (l2 < F)).astype(jnp.float32)
    q = jnp.dot(m.astype(jnp.float32), blockdiag,
                preferred_element_type=jnp.float32)
    r = jnp.roll(q, F, axis=1)
    ng = rows // F
    q3 = q.reshape(ng, F, D)
    r3 = r.reshape(ng, F, D)
    sub = lax.broadcasted_iota(jnp.int32, q3.shape, 1)
    lane = lax.broadcasted_iota(jnp.int32, q3.shape, 2)
    on_diag = sub == lane // 2
    par_hi = (lane % 2) == (lane // F)
    eq = (on_diag & par_hi).astype(jnp.float32)
    er = (on_diag & ~par_hi).astype(jnp.float32)
    counts_ref[...] = jnp.sum(q3 * eq + r3 * er, axis=1).astype(jnp.int32)

    @pl.when(pl.program_id(0) == 0)
    def _():
        r = lax.broadcasted_iota(jnp.int32, (TPAD, D), 0).astype(
            jnp.float32) * (1.0 / F)
        x = r * w1_ref[...] + b1_ref[...]
        h = x * jax.nn.sigmoid(x)
        ratio = jnp.dot(h, w2_ref[...], preferred_element_type=jnp.float32)
        e0 = emb_ref[0:1, :]
        e1 = emb_ref[1:2, :]
        table_ref[...] = ratio + b2_ref[...] + e0 + r * (e1 - e0)


def _counts_and_table(mask2d, emb_table, W1, b1, W2, b2):
    return pl.pallas_call(
        _tc_body,
        grid=(1,),
        in_specs=[
            pl.BlockSpec((N // 2, D), lambda i: (0, 0)),
            pl.BlockSpec((2, D), lambda i: (0, 0)),
            pl.BlockSpec((1, D), lambda i: (0, 0)),
            pl.BlockSpec((1, D), lambda i: (0, 0)),
            pl.BlockSpec((D, D), lambda i: (0, 0)),
            pl.BlockSpec((1, D), lambda i: (0, 0)),
        ],
        out_specs=[
            pl.BlockSpec((N // D, D), lambda i: (0, 0)),
            pl.BlockSpec((TPAD, D), lambda i: (0, 0)),
        ],
        out_shape=[
            jax.ShapeDtypeStruct((N // D, D), jnp.int32),
            jax.ShapeDtypeStruct((TPAD, D), jnp.float32),
        ],
    )(mask2d, emb_table, W1, b1, W2, b2)


NCHUNK = B_PER_W // CHUNK


def _sc_gather_body(table_hbm, counts_hbm, out_hbm, table_sp, idx_v,
                    rows, semg, semw):
    sid = lax.axis_index("s")
    wid = sid * NC + lax.axis_index("c")

    @pl.when(sid == 0)
    def _():
        pltpu.sync_copy(table_hbm, table_sp)

    pltpu.sync_copy(counts_hbm.at[pl.ds(wid * NCHUNK, NCHUNK)], idx_v)
    plsc.subcore_barrier()
    gathers = [
        pltpu.async_copy(table_sp.at[idx_v.at[k]], rows.at[k], semg)
        for k in range(NCHUNK)
    ]
    writes = []
    for k in range(NCHUNK):
        gathers[k].wait()
        writes.append(pltpu.async_copy(
            rows.at[k], out_hbm.at[pl.ds(wid * B_PER_W + k * CHUNK, CHUNK)],
            semw))
    for w in writes:
        w.wait()


@functools.cache
def _sc_gather():
    return pl.kernel(
        _sc_gather_body,
        out_type=jax.ShapeDtypeStruct((N, D), jnp.float32),
        mesh=plsc.VectorSubcoreMesh(core_axis_name="c", subcore_axis_name="s",
                                    num_cores=NC, num_subcores=NS),
        scratch_types=[
            pltpu.VMEM_SHARED((TPAD, D), jnp.float32),
            pltpu.VMEM((NCHUNK, CHUNK), jnp.int32),
            pltpu.VMEM((NCHUNK, CHUNK, D), jnp.float32),
            pltpu.SemaphoreType.DMA,
            pltpu.SemaphoreType.DMA,
        ],
    )


def kernel(mask, emb_table, W1, b1, W2, b2):
    Bsz, S, _ = mask.shape
    mask2d = mask.reshape(N // 2, D)
    counts, table = _counts_and_table(
        mask2d, emb_table, W1, b1.reshape(1, D), W2, b2.reshape(1, D))
    out = _sc_gather()(table, counts.reshape(N // CHUNK, CHUNK))
    return out.reshape(Bsz, S, D)
